# baseline (speedup 1.0000x reference)
import functools

import jax
import jax.numpy as jnp
from jax import lax
from jax.experimental import pallas as pl
from jax.experimental.pallas import tpu as pltpu
from jax.experimental.pallas import tpu_sc as plsc


def kernel(indices, values):
    B, H = indices.shape
    V, D = values.shape

    info = plsc.get_sparse_core_info()
    NC, NS = info.num_cores, info.num_subcores
    NW = NC * NS
    b_per_w = B // NW
    n_chunks = b_per_w
    NBUF = 8
    n_outer = n_chunks // NBUF
    LANES = 128

    HP = 56
    idxp = jnp.pad(indices.astype(jnp.int32), ((0, 0), (0, LANES - H)))

    @functools.partial(
        pl.kernel,
        mesh=plsc.VectorSubcoreMesh(core_axis_name="c", subcore_axis_name="s"),
        out_type=jax.ShapeDtypeStruct((B, H, D), jnp.float32),
        scratch_types=[
            pltpu.VMEM((b_per_w, LANES), jnp.int32),
            pltpu.VMEM((NBUF, HP, D), jnp.float32),
        ]
        + [pltpu.SemaphoreType.DMA] * (2 * NBUF),
        compiler_params=pltpu.CompilerParams(use_tc_tiling_on_sc=False),
    )
    def gather_kernel(table_hbm, idx_hbm, out_hbm, idx_v, rows_v, *sems):
        gsem = sems[:NBUF]
        wsem = sems[NBUF:]
        wid = lax.axis_index("s") * NC + lax.axis_index("c")
        base_b = wid * b_per_w

        def gather_start(i, k):
            pltpu.async_copy(
                table_hbm.at[idx_v.at[i, pl.ds(0, HP)]], rows_v.at[k], gsem[k]
            )

        def gather_wait(i, k):
            pltpu.make_async_copy(
                table_hbm.at[idx_v.at[i, pl.ds(0, HP)]], rows_v.at[k], gsem[k]
            ).wait()

        def write_start(i, k):
            pltpu.async_copy(
                rows_v.at[k, pl.ds(0, H), :], out_hbm.at[base_b + i], wsem[k]
            )

        def write_wait(k):
            pltpu.make_async_copy(
                rows_v.at[k, pl.ds(0, H), :], out_hbm.at[base_b], wsem[k]
            ).wait()

        pltpu.sync_copy(idx_hbm.at[pl.ds(base_b, b_per_w), :], idx_v)

        SLACK = 2
        for k in range(NBUF - SLACK):
            gather_start(k, k)

        def step(i, k, first):
            gather_wait(i, k)
            write_start(i, k)
            gb = (k - SLACK) % NBUF
            if not (first and k < SLACK):
                write_wait(gb)
            gather_start(i + NBUF - SLACK, gb)

        for k in range(NBUF):
            step(k, k, True)

        def outer(o, carry):
            for k in range(NBUF):
                step(o * NBUF + k, k, False)
            return carry

        lax.fori_loop(1, n_outer - 1, outer, 0)

        for k in range(NBUF):
            i = (n_outer - 1) * NBUF + k
            gather_wait(i, k)
            write_start(i, k)
            if k < SLACK:
                gb = (k - SLACK) % NBUF
                write_wait(gb)
                gather_start(i + NBUF - SLACK, gb)
        for k in range(NBUF):
            write_wait(k)

    return gather_kernel(values, idxp)
